# baseline (speedup 1.0000x reference)
import functools

import jax
import jax.numpy as jnp
from jax import lax
from jax.experimental import pallas as pl
from jax.experimental.pallas import tpu as pltpu
from jax.experimental.pallas import tpu_sc as plsc

NC, NS, L = 2, 16, 16
N = 10000
N_PAD = 10240
C = 128
E = 160000
M = 2500
M_PAD = 2560
NNZ_PAD = 7680

_MESH = plsc.VectorSubcoreMesh(
    core_axis_name="c", subcore_axis_name="s", num_cores=NC, num_subcores=NS
)
_SC_PARAMS = pltpu.CompilerParams(needs_layout_passes=False)

_MAGIC = 0x5F3759DF


def _rsqrt16(x):
    i = plsc.bitcast(x, jnp.int32)
    y = plsc.bitcast(_MAGIC - lax.shift_right_logical(i, 1), jnp.float32)
    for _ in range(3):
        y = y * (1.5 - 0.5 * x * y * y)
    return jnp.where(x > 0.5, y, 0.0)


def _copy16(dst, dst_off, src, src_off, n16, add=None):
    for g in range(n16):
        v = src[pl.ds(src_off + g * L, L)]
        if add is not None:
            v = v + add
        dst[pl.ds(dst_off + g * L, L)] = v


_CH = 80
_NCH = (E // NS) // _CH


def _cheb_body(src_hbm, packed_hbm, p1_hbm, p2_hbm, deg_sh, acc_sh,
               dfull, ones, degv,
               p0, p1, p2, rv0, rv1, rv2, cv0, cv1, cv2, vv0, vv1, vv2,
               rows0, rows1, rows2, sp0, sp1, sp2, sg0, sg1, sg2, ss0, ss1, ss2):
    c = lax.axis_index("c")
    s = lax.axis_index("s")
    coff = c * N_PAD
    ch, nch = _CH, _NCH
    nsl = 640
    arows = N_PAD // NS
    nb = arows // 80
    P = [(p0, sp0), (p1, sp1), (p2, sp2)]
    R = [(rv0, cv0, vv0, rows0, sg0, ss0),
         (rv1, cv1, vv1, rows1, sg1, ss1),
         (rv2, cv2, vv2, rows2, sg2, ss2)]

    def pslice(t):
        return packed_hbm.at[pl.ds((s * nch + t) * 2 * ch, 2 * ch)]

    def pstart(t, b):
        pp, sp = P[b]
        pltpu.async_copy(pslice(t), pp, sp)

    def pwait(t, b):
        pp, sp = P[b]
        pltpu.make_async_copy(pslice(t), pp, sp).wait()

    def zero_acc():
        def zero_row(i, _):
            for j in range(C // L):
                rows0[i, pl.ds(j * L, L)] = jnp.zeros((L,), jnp.float32)
            return _
        lax.fori_loop(0, 80, zero_row, None)
        for k in range(nb):
            pltpu.async_copy(rows0, acc_sh.at[pl.ds(s * arows + k * 80, 80)], sg0)
        for k in range(nb):
            pltpu.make_async_copy(rows0, acc_sh.at[pl.ds(s * arows + k * 80, 80)], sg0).wait()

    def zdeg16(i, _):
        degv[pl.ds(i * L, L)] = jnp.zeros((L,), jnp.float32)
        return _
    lax.fori_loop(0, nsl // L, zdeg16, None)
    pltpu.sync_copy(degv, deg_sh.at[pl.ds(s * nsl, nsl)])
    for g in range(ch // L):
        ones[pl.ds(g * L, L)] = jnp.ones((L,), jnp.float32)
    zero_acc()
    plsc.subcore_barrier()

    def dextract(b):
        pp, _ = P[b]
        rv = R[b][0]
        for g in range(ch // L):
            rv[pl.ds(g * L, L)] = pp[pl.ds(g * L, L)]

    def dscat(b):
        rv, ss = R[b][0], R[b][5]
        pltpu.async_copy(ones, deg_sh.at[rv], ss, add=True)

    def dswait(b):
        rv, ss = R[b][0], R[b][5]
        pltpu.make_async_copy(ones, deg_sh.at[rv], ss).wait()

    pstart(0, 0)
    pstart(1, 1)
    pwait(0, 0); dextract(0); pstart(2, 0); dscat(0)
    pwait(1, 1); dextract(1); pstart(3, 1); dscat(1)

    def deg_step(g, _):
        t0 = 2 * g
        pwait(t0, 0); dswait(0); dextract(0); pstart(t0 + 2, 0); dscat(0)
        pwait(t0 + 1, 1); dswait(1); dextract(1); pstart(t0 + 3, 1); dscat(1)
        return _
    lax.fori_loop(1, (_NCH - 1) // 2, deg_step, None)
    pwait(nch - 1, 0); dswait(0); dextract(0); dscat(0)
    dswait(0); dswait(1)
    pwait(nch, 1)
    plsc.subcore_barrier()

    pltpu.sync_copy(deg_sh.at[pl.ds(s * nsl, nsl)], degv)

    def dinv16(g, _):
        degv[pl.ds(g * L, L)] = _rsqrt16(degv[pl.ds(g * L, L)])
        return _
    lax.fori_loop(0, nsl // L, dinv16, None)
    pltpu.sync_copy(degv, deg_sh.at[pl.ds(s * nsl, nsl)])
    plsc.subcore_barrier()
    pltpu.sync_copy(deg_sh, dfull)

    def prop(src, out):
        def gstart(t, b):
            pwait(t, b)
            pp, _ = P[b]
            rv, cv, vv, rows, sg, _ = R[b]
            for g in range(ch // L):
                sl = pl.ds(g * L, L)
                rvv = pp[pl.ds(g * L, L)]
                cvv = pp[pl.ds(ch + g * L, L)]
                rv[sl] = rvv
                cv[sl] = cvv + coff
                vv[sl] = -(plsc.load_gather(dfull, [rvv]) *
                           plsc.load_gather(dfull, [cvv]))
            pltpu.async_copy(src.at[cv], rows, sg)
            pstart(t + 2, (b + 2) % 3)

        def gwait(b):
            _, cv, _, rows, sg, _ = R[b]
            pltpu.make_async_copy(src.at[cv], rows, sg).wait()

        def scale(b):
            _, _, vv, rows, _, _ = R[b]

            @plsc.parallel_loop(0, ch, 1, unroll=4)
            def srow(i):
                nb_ = plsc.load_gather(vv, [jnp.full((L,), i, jnp.int32)])
                for j in range(C // L):
                    sl = (i, pl.ds(j * L, L))
                    rows[sl] = rows[sl] * nb_

        def sstart(b):
            rv, _, _, rows, _, ss = R[b]
            pltpu.async_copy(rows, acc_sh.at[rv], ss, add=True)

        def swait(b):
            rv, _, _, rows, _, ss = R[b]
            pltpu.make_async_copy(rows, acc_sh.at[rv], ss).wait()

        pstart(0, 0)
        pstart(1, 1)
        gstart(0, 0)
        gstart(1, 1)
        gwait(0); scale(0); sstart(0); gstart(2, 2)

        def chunk(t, b, bprev):
            gwait(b); scale(b); sstart(b); swait(bprev); gstart(t + 2, (b + 2) % 3)

        def step(g, _):
            t0 = 3 * g + 1
            chunk(t0, 1, 0)
            chunk(t0 + 1, 2, 1)
            chunk(t0 + 2, 0, 2)
            return _
        lax.fori_loop(0, (nch - 2) // 3, step, None)
        b_last = (nch - 1) % 3
        gwait(b_last); scale(b_last); sstart(b_last); swait((nch - 2) % 3)
        swait(b_last)
        gwait(nch % 3)
        pwait(nch + 1, (nch + 1) % 3)
        pwait(nch + 2, (nch + 2) % 3)
        plsc.subcore_barrier()
        r = s * arows
        pltpu.sync_copy(acc_sh.at[pl.ds(r, arows)],
                        out.at[pl.ds(c * N_PAD + r, arows)])

    prop(src_hbm, p1_hbm)
    zero_acc()
    plsc.subcore_barrier()
    prop(p1_hbm, p2_hbm)


_cheb_kernel = pl.kernel(
    _cheb_body,
    out_type=(jax.ShapeDtypeStruct((NC * N_PAD, C), jnp.float32),
              jax.ShapeDtypeStruct((NC * N_PAD, C), jnp.float32)),
    mesh=_MESH,
    compiler_params=_SC_PARAMS,
    scratch_types=(
        [pltpu.VMEM_SHARED((NS * 640,), jnp.float32),
         pltpu.VMEM_SHARED((N_PAD, C), jnp.float32),
         pltpu.VMEM((NS * 640,), jnp.float32),
         pltpu.VMEM((_CH,), jnp.float32),
         pltpu.VMEM((640,), jnp.float32)]
        + [pltpu.VMEM((2 * _CH,), jnp.int32)] * 3
        + [pltpu.VMEM((_CH,), jnp.int32)] * 3
        + [pltpu.VMEM((_CH,), jnp.int32)] * 3
        + [pltpu.VMEM((_CH,), jnp.float32)] * 3
        + [pltpu.VMEM((_CH, C), jnp.float32)] * 3
        + [pltpu.SemaphoreType.DMA] * 9
    ),
)


def _pack_chunks(dst, srcidx, val, ch):
    q = dst.shape[0] // ch
    packed = jnp.concatenate([
        dst.reshape(q, ch),
        srcidx.reshape(q, ch),
        lax.bitcast_convert_type(val, jnp.int32).reshape(q, ch),
    ], axis=1)
    return jnp.pad(packed, ((0, 3), (0, 0))).reshape(-1)


def _make_scatter(n_acc, n_entries, n_out_rows, ch):
    eps = n_entries // NS
    nch = eps // ch
    arows = n_acc // NS
    assert eps % ch == 0 and (nch - 2) % 3 == 0 and nch >= 5
    assert ch % L == 0 and ch <= 128
    assert arows % 80 == 0 and ch >= 80
    nb = arows // 80

    def body(src_hbm, packed_hbm, out_hbm, acc_sh,
             p0, p1, p2, rv0, rv1, rv2, cv0, cv1, cv2, vv0, vv1, vv2,
             rows0, rows1, rows2, sp0, sp1, sp2, sg0, sg1, sg2, ss0, ss1, ss2):
        c = lax.axis_index("c")
        s = lax.axis_index("s")
        coff = c * N_PAD
        P = [(p0, sp0), (p1, sp1), (p2, sp2)]
        R = [(rv0, cv0, vv0, rows0, sg0, ss0),
             (rv1, cv1, vv1, rows1, sg1, ss1),
             (rv2, cv2, vv2, rows2, sg2, ss2)]

        def zero_row(i, _):
            for j in range(C // L):
                rows0[i, pl.ds(j * L, L)] = jnp.zeros((L,), jnp.float32)
            return _
        lax.fori_loop(0, 80, zero_row, None)
        zsrc = rows0 if ch == 80 else rows0.at[pl.ds(0, 80)]
        for k in range(nb):
            pltpu.async_copy(zsrc, acc_sh.at[pl.ds(s * arows + k * 80, 80)], sg0)
        for k in range(nb):
            pltpu.make_async_copy(zsrc, acc_sh.at[pl.ds(s * arows + k * 80, 80)], sg0).wait()
        plsc.subcore_barrier()

        def pslice(t):
            return packed_hbm.at[pl.ds((s * nch + t) * 3 * ch, 3 * ch)]

        def pstart(t, b):
            p, sp = P[b]
            pltpu.async_copy(pslice(t), p, sp)

        def pwait(t, b):
            p, sp = P[b]
            pltpu.make_async_copy(pslice(t), p, sp).wait()

        def gstart(t, b):
            pwait(t, b)
            p, _ = P[b]
            rv, cv, vv, rows, sg, _ = R[b]
            for g in range(ch // L):
                sl = pl.ds(g * L, L)
                rv[sl] = p[pl.ds(g * L, L)]
                cv[sl] = p[pl.ds(ch + g * L, L)] + coff
                vv[sl] = plsc.bitcast(p[pl.ds(2 * ch + g * L, L)], jnp.float32)
            pltpu.async_copy(src_hbm.at[cv], rows, sg)
            pstart(t + 2, (b + 2) % 3)

        def gwait(b):
            _, cv, _, rows, sg, _ = R[b]
            pltpu.make_async_copy(src_hbm.at[cv], rows, sg).wait()

        def scale(b):
            _, _, vv, rows, _, _ = R[b]

            @plsc.parallel_loop(0, ch, 1, unroll=4)
            def srow(i):
                nb = plsc.load_gather(vv, [jnp.full((L,), i, jnp.int32)])
                for j in range(C // L):
                    sl = (i, pl.ds(j * L, L))
                    rows[sl] = rows[sl] * nb

        def sstart(b):
            rv, _, _, rows, _, ss = R[b]
            pltpu.async_copy(rows, acc_sh.at[rv], ss, add=True)

        def swait(b):
            rv, _, _, rows, _, ss = R[b]
            pltpu.make_async_copy(rows, acc_sh.at[rv], ss).wait()

        pstart(0, 0)
        pstart(1, 1)
        gstart(0, 0)
        gstart(1, 1)
        gwait(0); scale(0); sstart(0); gstart(2, 2)

        def chunk(t, b, bprev):
            gwait(b); scale(b); sstart(b); swait(bprev); gstart(t + 2, (b + 2) % 3)

        def step(g, _):
            t0 = 3 * g + 1
            chunk(t0, 1, 0)
            chunk(t0 + 1, 2, 1)
            chunk(t0 + 2, 0, 2)
            return _
        lax.fori_loop(0, (nch - 2) // 3, step, None)
        b_last = (nch - 1) % 3
        gwait(b_last); scale(b_last); sstart(b_last); swait((nch - 2) % 3)
        swait(b_last)
        gwait(nch % 3)
        pwait(nch + 1, (nch + 1) % 3)
        pwait(nch + 2, (nch + 2) % 3)
        plsc.subcore_barrier()

        r = s * arows
        pltpu.sync_copy(acc_sh.at[pl.ds(r, arows)],
                        out_hbm.at[pl.ds(c * n_out_rows + r, arows)])

    return pl.kernel(
        body,
        out_type=jax.ShapeDtypeStruct((NC * n_out_rows, C), jnp.float32),
        mesh=_MESH,
        compiler_params=_SC_PARAMS,
        scratch_types=(
            [pltpu.VMEM_SHARED((n_acc, C), jnp.float32)]
            + [pltpu.VMEM((3 * ch,), jnp.int32)] * 3
            + [pltpu.VMEM((ch,), jnp.int32)] * 3
            + [pltpu.VMEM((ch,), jnp.int32)] * 3
            + [pltpu.VMEM((ch,), jnp.float32)] * 3
            + [pltpu.VMEM((ch, C), jnp.float32)] * 3
            + [pltpu.SemaphoreType.DMA] * 9
        ),
    )


_pool_kernel = _make_scatter(n_acc=M_PAD, n_entries=NNZ_PAD, n_out_rows=M_PAD,
                             ch=96)


def _mm_body(x_ref, p1_ref, p2_ref, w_ref, b_ref, o_ref):
    w0 = w_ref[0] - w_ref[2]
    w1 = w_ref[1]
    w2 = 2.0 * w_ref[2]
    z = jnp.dot(x_ref[...], w0, preferred_element_type=jnp.float32)
    z = z + jnp.dot(p1_ref[...], w1, preferred_element_type=jnp.float32)
    z = z + jnp.dot(p2_ref[...], w2, preferred_element_type=jnp.float32)
    z = z + b_ref[...]
    o_ref[...] = jnp.where(z > 0, z, jnp.exp(z) - 1.0)


def _mm_call(xf, p1f, p2f, W, b):
    BN = xf.shape[0]
    blk = 1024
    grid = BN // blk
    row_spec = pl.BlockSpec((blk, C), lambda i: (i, 0))
    return pl.pallas_call(
        _mm_body,
        grid=(grid,),
        in_specs=[row_spec, row_spec, row_spec,
                  pl.BlockSpec((3, C, C), lambda i: (0, 0, 0)),
                  pl.BlockSpec((1, C), lambda i: (0, 0))],
        out_specs=row_spec,
        out_shape=jax.ShapeDtypeStruct((BN, C), jnp.float32),
    )(xf, p1f, p2f, W, b)


def _pack2_chunks(dst, srcidx, ch):
    q = dst.shape[0] // ch
    packed = jnp.concatenate([dst.reshape(q, ch), srcidx.reshape(q, ch)], axis=1)
    return jnp.pad(packed, ((0, 3), (0, 0))).reshape(-1)


def kernel(x, edge_index, trans_row, trans_col, trans_value, W, b):
    B = x.shape[0]
    row = edge_index[0]
    col = edge_index[1]

    xf = jnp.pad(x, ((0, 0), (0, N_PAD - N), (0, 0))).reshape(B * N_PAD, C)
    epack = _pack2_chunks(row, col, _CH)
    p1f, p2f = _cheb_kernel(xf, epack)

    hf = _mm_call(xf, p1f, p2f, W, b.reshape(1, C))

    pad = NNZ_PAD - trans_row.shape[0]
    tr = jnp.concatenate([trans_row, jnp.zeros((pad,), jnp.int32)])
    tc = jnp.concatenate([trans_col, jnp.zeros((pad,), jnp.int32)])
    tv = jnp.concatenate([trans_value, jnp.zeros((pad,), jnp.float32)])

    pooled = _pool_kernel(hf, _pack_chunks(tr, tc, tv, 96))
    return pooled.reshape(B, M_PAD, C)[:, :M, :]


# speedup vs baseline: 1.0129x; 1.0129x over previous
import functools

import jax
import jax.numpy as jnp
from jax import lax
from jax.experimental import pallas as pl
from jax.experimental.pallas import tpu as pltpu
from jax.experimental.pallas import tpu_sc as plsc

NC, NS, L = 2, 16, 16
N = 10000
N_PAD = 10240
C = 128
E = 160000
M = 2500
M_PAD = 2560
NNZ_PAD = 7680

_MESH = plsc.VectorSubcoreMesh(
    core_axis_name="c", subcore_axis_name="s", num_cores=NC, num_subcores=NS
)
_SC_PARAMS = pltpu.CompilerParams(needs_layout_passes=False)

_MAGIC = 0x5F3759DF


def _rsqrt16(x):
    i = plsc.bitcast(x, jnp.int32)
    y = plsc.bitcast(_MAGIC - lax.shift_right_logical(i, 1), jnp.float32)
    for _ in range(3):
        y = y * (1.5 - 0.5 * x * y * y)
    return jnp.where(x > 0.5, y, 0.0)


def _copy16(dst, dst_off, src, src_off, n16, add=None):
    for g in range(n16):
        v = src[pl.ds(src_off + g * L, L)]
        if add is not None:
            v = v + add
        dst[pl.ds(dst_off + g * L, L)] = v


_CH = 80
_NCH = (E // NS) // _CH


def _cheb_body(src_hbm, packed_hbm, p1_hbm, p2_hbm, deg_sh, acc_sh,
               dfull, ones, degv,
               p0, p1, p2, rv0, rv1, rv2, cv0, cv1, cv2, vv0, vv1, vv2,
               rows0, rows1, rows2, sp0, sp1, sp2, sg0, sg1, sg2, ss0, ss1, ss2):
    c = lax.axis_index("c")
    s = lax.axis_index("s")
    ch, nch = _CH, _NCH
    nsl = 640
    arows = N_PAD // NS
    nb = arows // 80
    P = [(p0, sp0), (p1, sp1), (p2, sp2)]
    R = [(rv0, cv0, vv0, rows0, sg0, ss0),
         (rv1, cv1, vv1, rows1, sg1, ss1),
         (rv2, cv2, vv2, rows2, sg2, ss2)]

    def pslice(t):
        return packed_hbm.at[pl.ds((s * nch + t) * 2 * ch, 2 * ch)]

    def pstart(t, b):
        pp, sp = P[b]
        pltpu.async_copy(pslice(t), pp, sp)

    def pwait(t, b):
        pp, sp = P[b]
        pltpu.make_async_copy(pslice(t), pp, sp).wait()

    def zero_acc():
        def zero_row(i, _):
            for j in range(C // L):
                rows0[i, pl.ds(j * L, L)] = jnp.zeros((L,), jnp.float32)
            return _
        lax.fori_loop(0, 80, zero_row, None)
        for k in range(nb):
            pltpu.async_copy(rows0, acc_sh.at[pl.ds(s * arows + k * 80, 80)], sg0)
        for k in range(nb):
            pltpu.make_async_copy(rows0, acc_sh.at[pl.ds(s * arows + k * 80, 80)], sg0).wait()

    def zdeg16(i, _):
        degv[pl.ds(i * L, L)] = jnp.zeros((L,), jnp.float32)
        return _
    lax.fori_loop(0, nsl // L, zdeg16, None)
    pltpu.sync_copy(degv, deg_sh.at[pl.ds(s * nsl, nsl)])
    for g in range(ch // L):
        ones[pl.ds(g * L, L)] = jnp.ones((L,), jnp.float32)
    zero_acc()
    plsc.subcore_barrier()

    def dextract(b):
        pp, _ = P[b]
        rv = R[b][0]
        for g in range(ch // L):
            rv[pl.ds(g * L, L)] = pp[pl.ds(g * L, L)]

    def dscat(b):
        rv, ss = R[b][0], R[b][5]
        pltpu.async_copy(ones, deg_sh.at[rv], ss, add=True)

    def dswait(b):
        rv, ss = R[b][0], R[b][5]
        pltpu.make_async_copy(ones, deg_sh.at[rv], ss).wait()

    pstart(0, 0)
    pstart(1, 1)
    pwait(0, 0); dextract(0); pstart(2, 0); dscat(0)
    pwait(1, 1); dextract(1); pstart(3, 1); dscat(1)

    def deg_step(g, _):
        t0 = 2 * g
        pwait(t0, 0); dswait(0); dextract(0); pstart(t0 + 2, 0); dscat(0)
        pwait(t0 + 1, 1); dswait(1); dextract(1); pstart(t0 + 3, 1); dscat(1)
        return _
    lax.fori_loop(1, (_NCH - 1) // 2, deg_step, None)
    pwait(nch - 1, 0); dswait(0); dextract(0); dscat(0)
    dswait(0); dswait(1)
    pwait(nch, 1)
    plsc.subcore_barrier()

    pltpu.sync_copy(deg_sh.at[pl.ds(s * nsl, nsl)], degv)

    def dinv16(g, _):
        degv[pl.ds(g * L, L)] = _rsqrt16(degv[pl.ds(g * L, L)])
        return _
    lax.fori_loop(0, nsl // L, dinv16, None)
    pltpu.sync_copy(degv, deg_sh.at[pl.ds(s * nsl, nsl)])
    plsc.subcore_barrier()
    pltpu.sync_copy(deg_sh, dfull)

    def prop(src, out, coff):
        def gstart(t, b):
            pwait(t, b)
            pp, _ = P[b]
            rv, cv, vv, rows, sg, _ = R[b]
            for g in range(ch // L):
                sl = pl.ds(g * L, L)
                rvv = pp[pl.ds(g * L, L)]
                cvv = pp[pl.ds(ch + g * L, L)]
                rv[sl] = rvv
                cv[sl] = cvv + coff
                vv[sl] = -(plsc.load_gather(dfull, [rvv]) *
                           plsc.load_gather(dfull, [cvv]))
            pltpu.async_copy(src.at[cv], rows, sg)
            pstart(t + 2, (b + 2) % 3)

        def gwait(b):
            _, cv, _, rows, sg, _ = R[b]
            pltpu.make_async_copy(src.at[cv], rows, sg).wait()

        def scale(b):
            _, _, vv, rows, _, _ = R[b]

            @plsc.parallel_loop(0, ch, 1, unroll=4)
            def srow(i):
                nb_ = plsc.load_gather(vv, [jnp.full((L,), i, jnp.int32)])
                for j in range(C // L):
                    sl = (i, pl.ds(j * L, L))
                    rows[sl] = rows[sl] * nb_

        def sstart(b):
            rv, _, _, rows, _, ss = R[b]
            pltpu.async_copy(rows, acc_sh.at[rv], ss, add=True)

        def swait(b):
            rv, _, _, rows, _, ss = R[b]
            pltpu.make_async_copy(rows, acc_sh.at[rv], ss).wait()

        pstart(0, 0)
        pstart(1, 1)
        gstart(0, 0)
        gstart(1, 1)
        gwait(0); scale(0); sstart(0); gstart(2, 2)

        def chunk(t, b, bprev):
            gwait(b); scale(b); sstart(b); swait(bprev); gstart(t + 2, (b + 2) % 3)

        def step(g, _):
            t0 = 3 * g + 1
            chunk(t0, 1, 0)
            chunk(t0 + 1, 2, 1)
            chunk(t0 + 2, 0, 2)
            return _
        lax.fori_loop(0, (nch - 2) // 3, step, None)
        b_last = (nch - 1) % 3
        gwait(b_last); scale(b_last); sstart(b_last); swait((nch - 2) % 3)
        swait(b_last)
        gwait(nch % 3)
        pwait(nch + 1, (nch + 1) % 3)
        pwait(nch + 2, (nch + 2) % 3)
        plsc.subcore_barrier()
        r = s * arows
        pltpu.sync_copy(acc_sh.at[pl.ds(r, arows)],
                        out.at[pl.ds(c * N_PAD + r, arows)])

    prop(src_hbm, p1_hbm, c * N)
    zero_acc()
    plsc.subcore_barrier()
    prop(p1_hbm, p2_hbm, c * N_PAD)


_cheb_kernel = pl.kernel(
    _cheb_body,
    out_type=(jax.ShapeDtypeStruct((NC * N_PAD, C), jnp.float32),
              jax.ShapeDtypeStruct((NC * N_PAD, C), jnp.float32)),
    mesh=_MESH,
    compiler_params=_SC_PARAMS,
    scratch_types=(
        [pltpu.VMEM_SHARED((NS * 640,), jnp.float32),
         pltpu.VMEM_SHARED((N_PAD, C), jnp.float32),
         pltpu.VMEM((NS * 640,), jnp.float32),
         pltpu.VMEM((_CH,), jnp.float32),
         pltpu.VMEM((640,), jnp.float32)]
        + [pltpu.VMEM((2 * _CH,), jnp.int32)] * 3
        + [pltpu.VMEM((_CH,), jnp.int32)] * 3
        + [pltpu.VMEM((_CH,), jnp.int32)] * 3
        + [pltpu.VMEM((_CH,), jnp.float32)] * 3
        + [pltpu.VMEM((_CH, C), jnp.float32)] * 3
        + [pltpu.SemaphoreType.DMA] * 9
    ),
)


def _pack_chunks(dst, srcidx, val, ch):
    q = dst.shape[0] // ch
    packed = jnp.concatenate([
        dst.reshape(q, ch),
        srcidx.reshape(q, ch),
        lax.bitcast_convert_type(val, jnp.int32).reshape(q, ch),
    ], axis=1)
    return jnp.pad(packed, ((0, 3), (0, 0))).reshape(-1)


def _make_scatter(n_acc, n_entries, n_out_rows, ch, src_stride):
    eps = n_entries // NS
    nch = eps // ch
    arows = n_acc // NS
    assert eps % ch == 0 and (nch - 2) % 3 == 0 and nch >= 5
    assert ch % L == 0 and ch <= 128
    assert arows % 80 == 0 and ch >= 80
    nb = arows // 80

    def body(src_hbm, packed_hbm, out_hbm, acc_sh,
             p0, p1, p2, rv0, rv1, rv2, cv0, cv1, cv2, vv0, vv1, vv2,
             rows0, rows1, rows2, sp0, sp1, sp2, sg0, sg1, sg2, ss0, ss1, ss2):
        c = lax.axis_index("c")
        s = lax.axis_index("s")
        coff = c * src_stride
        P = [(p0, sp0), (p1, sp1), (p2, sp2)]
        R = [(rv0, cv0, vv0, rows0, sg0, ss0),
             (rv1, cv1, vv1, rows1, sg1, ss1),
             (rv2, cv2, vv2, rows2, sg2, ss2)]

        def zero_row(i, _):
            for j in range(C // L):
                rows0[i, pl.ds(j * L, L)] = jnp.zeros((L,), jnp.float32)
            return _
        lax.fori_loop(0, 80, zero_row, None)
        zsrc = rows0 if ch == 80 else rows0.at[pl.ds(0, 80)]
        for k in range(nb):
            pltpu.async_copy(zsrc, acc_sh.at[pl.ds(s * arows + k * 80, 80)], sg0)
        for k in range(nb):
            pltpu.make_async_copy(zsrc, acc_sh.at[pl.ds(s * arows + k * 80, 80)], sg0).wait()
        plsc.subcore_barrier()

        def pslice(t):
            return packed_hbm.at[pl.ds((s * nch + t) * 3 * ch, 3 * ch)]

        def pstart(t, b):
            p, sp = P[b]
            pltpu.async_copy(pslice(t), p, sp)

        def pwait(t, b):
            p, sp = P[b]
            pltpu.make_async_copy(pslice(t), p, sp).wait()

        def gstart(t, b):
            pwait(t, b)
            p, _ = P[b]
            rv, cv, vv, rows, sg, _ = R[b]
            for g in range(ch // L):
                sl = pl.ds(g * L, L)
                rv[sl] = p[pl.ds(g * L, L)]
                cv[sl] = p[pl.ds(ch + g * L, L)] + coff
                vv[sl] = plsc.bitcast(p[pl.ds(2 * ch + g * L, L)], jnp.float32)
            pltpu.async_copy(src_hbm.at[cv], rows, sg)
            pstart(t + 2, (b + 2) % 3)

        def gwait(b):
            _, cv, _, rows, sg, _ = R[b]
            pltpu.make_async_copy(src_hbm.at[cv], rows, sg).wait()

        def scale(b):
            _, _, vv, rows, _, _ = R[b]

            @plsc.parallel_loop(0, ch, 1, unroll=4)
            def srow(i):
                nb = plsc.load_gather(vv, [jnp.full((L,), i, jnp.int32)])
                for j in range(C // L):
                    sl = (i, pl.ds(j * L, L))
                    rows[sl] = rows[sl] * nb

        def sstart(b):
            rv, _, _, rows, _, ss = R[b]
            pltpu.async_copy(rows, acc_sh.at[rv], ss, add=True)

        def swait(b):
            rv, _, _, rows, _, ss = R[b]
            pltpu.make_async_copy(rows, acc_sh.at[rv], ss).wait()

        pstart(0, 0)
        pstart(1, 1)
        gstart(0, 0)
        gstart(1, 1)
        gwait(0); scale(0); sstart(0); gstart(2, 2)

        def chunk(t, b, bprev):
            gwait(b); scale(b); sstart(b); swait(bprev); gstart(t + 2, (b + 2) % 3)

        def step(g, _):
            t0 = 3 * g + 1
            chunk(t0, 1, 0)
            chunk(t0 + 1, 2, 1)
            chunk(t0 + 2, 0, 2)
            return _
        lax.fori_loop(0, (nch - 2) // 3, step, None)
        b_last = (nch - 1) % 3
        gwait(b_last); scale(b_last); sstart(b_last); swait((nch - 2) % 3)
        swait(b_last)
        gwait(nch % 3)
        pwait(nch + 1, (nch + 1) % 3)
        pwait(nch + 2, (nch + 2) % 3)
        plsc.subcore_barrier()

        r = s * arows
        pltpu.sync_copy(acc_sh.at[pl.ds(r, arows)],
                        out_hbm.at[pl.ds(c * n_out_rows + r, arows)])

    return pl.kernel(
        body,
        out_type=jax.ShapeDtypeStruct((NC * n_out_rows, C), jnp.float32),
        mesh=_MESH,
        compiler_params=_SC_PARAMS,
        scratch_types=(
            [pltpu.VMEM_SHARED((n_acc, C), jnp.float32)]
            + [pltpu.VMEM((3 * ch,), jnp.int32)] * 3
            + [pltpu.VMEM((ch,), jnp.int32)] * 3
            + [pltpu.VMEM((ch,), jnp.int32)] * 3
            + [pltpu.VMEM((ch,), jnp.float32)] * 3
            + [pltpu.VMEM((ch, C), jnp.float32)] * 3
            + [pltpu.SemaphoreType.DMA] * 9
        ),
    )


_pool_kernel = _make_scatter(n_acc=M_PAD, n_entries=NNZ_PAD, n_out_rows=M_PAD,
                             ch=96, src_stride=N)


def _mm_body(x_ref, p1_ref, p2_ref, w_ref, b_ref, o_ref):
    w0 = w_ref[0] - w_ref[2]
    w1 = w_ref[1]
    w2 = 2.0 * w_ref[2]
    z = jnp.dot(x_ref[0], w0, preferred_element_type=jnp.float32)
    z = z + jnp.dot(p1_ref[0], w1, preferred_element_type=jnp.float32)
    z = z + jnp.dot(p2_ref[0], w2, preferred_element_type=jnp.float32)
    z = z + b_ref[...]
    o_ref[0] = jnp.where(z > 0, z, jnp.exp(z) - 1.0)


def _mm_call(x3, p13, p23, W, b):
    B = x3.shape[0]
    blk = 1000
    spec = pl.BlockSpec((1, blk, C), lambda bi, i: (bi, i, 0))
    return pl.pallas_call(
        _mm_body,
        grid=(B, N // blk),
        in_specs=[spec, spec, spec,
                  pl.BlockSpec((3, C, C), lambda bi, i: (0, 0, 0)),
                  pl.BlockSpec((1, C), lambda bi, i: (0, 0))],
        out_specs=spec,
        out_shape=jax.ShapeDtypeStruct((B, N, C), jnp.float32),
    )(x3, p13, p23, W, b)


def _pack2_chunks(dst, srcidx, ch):
    q = dst.shape[0] // ch
    packed = jnp.concatenate([dst.reshape(q, ch), srcidx.reshape(q, ch)], axis=1)
    return jnp.pad(packed, ((0, 3), (0, 0))).reshape(-1)


def kernel(x, edge_index, trans_row, trans_col, trans_value, W, b):
    B = x.shape[0]
    row = edge_index[0]
    col = edge_index[1]

    xf = x.reshape(B * N, C)
    epack = _pack2_chunks(row, col, _CH)
    p1f, p2f = _cheb_kernel(xf, epack)

    h3 = _mm_call(x, p1f.reshape(B, N_PAD, C), p2f.reshape(B, N_PAD, C),
                  W, b.reshape(1, C))
    hf = h3.reshape(B * N, C)

    pad = NNZ_PAD - trans_row.shape[0]
    tr = jnp.concatenate([trans_row, jnp.zeros((pad,), jnp.int32)])
    tc = jnp.concatenate([trans_col, jnp.zeros((pad,), jnp.int32)])
    tv = jnp.concatenate([trans_value, jnp.zeros((pad,), jnp.float32)])

    pooled = _pool_kernel(hf, _pack_chunks(tr, tc, tv, 96))
    return pooled.reshape(B, M_PAD, C)[:, :M, :]


# speedup vs baseline: 1.0474x; 1.0341x over previous
import jax
import jax.numpy as jnp
from jax import lax
from jax.experimental import pallas as pl
from jax.experimental.pallas import tpu as pltpu
from jax.experimental.pallas import tpu_sc as plsc

NC, NS, L = 2, 16, 16
N = 10000
N_PAD = 10240
C = 128
E = 160000
M = 2500
M_PAD = 2560
NNZ_PAD = 7680

_MESH = plsc.VectorSubcoreMesh(
    core_axis_name="c", subcore_axis_name="s", num_cores=NC, num_subcores=NS
)
_SC_PARAMS = pltpu.CompilerParams(needs_layout_passes=False)

_MAGIC = 0x5F3759DF


def _rsqrt16(x):
    i = plsc.bitcast(x, jnp.int32)
    y = plsc.bitcast(_MAGIC - lax.shift_right_logical(i, 1), jnp.float32)
    for _ in range(3):
        y = y * (1.5 - 0.5 * x * y * y)
    return jnp.where(x > 0.5, y, 0.0)


def _copy16(dst, dst_off, src, src_off, n16):
    for g in range(n16):
        dst[pl.ds(dst_off + g * L, L)] = src[pl.ds(src_off + g * L, L)]


def _norm_body(row_hbm, col_hbm, norm_hbm, deg_sh, dinv_sh, zb, ones,
               rowm, colm, rv0, rv1, dfull, nbuf, sd0, sd1):
    c = lax.axis_index("c")
    s = lax.axis_index("s")
    eps = E // NS
    nsl = 640
    ch = 80
    nch = eps // ch

    def zero16(i, _):
        zb[pl.ds(i * L, L)] = jnp.zeros((L,), jnp.float32)
        return _
    lax.fori_loop(0, nsl // L, zero16, None)
    for g in range(ch // L):
        ones[pl.ds(g * L, L)] = jnp.ones((L,), jnp.float32)
    pltpu.sync_copy(zb, deg_sh.at[pl.ds(s * nsl, nsl)])
    pltpu.sync_copy(row_hbm.at[pl.ds(s * eps, eps)], rowm)
    pltpu.sync_copy(col_hbm.at[pl.ds(s * eps, eps)], colm)
    plsc.subcore_barrier()

    def dstart(t, rv, sem):
        _copy16(rv, 0, rowm, t * ch, ch // L)
        pltpu.async_copy(ones, deg_sh.at[rv], sem, add=True)

    def dwait(rv, sem):
        pltpu.make_async_copy(ones, deg_sh.at[rv], sem).wait()

    dstart(0, rv0, sd0)
    dstart(1, rv1, sd1)

    def deg_step(g, _):
        t0 = 2 * g
        dwait(rv0, sd0)
        dstart(t0, rv0, sd0)
        dwait(rv1, sd1)
        dstart(t0 + 1, rv1, sd1)
        return _
    lax.fori_loop(1, nch // 2, deg_step, None)
    dwait(rv0, sd0)
    dstart(nch - 1, rv0, sd0)
    dwait(rv0, sd0)
    dwait(rv1, sd1)
    plsc.subcore_barrier()

    pltpu.sync_copy(deg_sh.at[pl.ds(s * nsl, nsl)], zb)

    def dinv16(g, _):
        zb[pl.ds(g * L, L)] = _rsqrt16(zb[pl.ds(g * L, L)])
        return _
    lax.fori_loop(0, nsl // L, dinv16, None)
    pltpu.sync_copy(zb, dinv_sh.at[pl.ds(s * nsl, nsl)])
    plsc.subcore_barrier()
    pltpu.sync_copy(dinv_sh, dfull)

    @pl.when(c == 0)
    def _():
        def norm16(k, _):
            sl = pl.ds(k * L, L)
            dr = plsc.load_gather(dfull, [rowm[sl]])
            dc = plsc.load_gather(dfull, [colm[sl]])
            nbuf[sl] = -(dr * dc)
            return _
        lax.fori_loop(0, eps // L, norm16, None)
        pltpu.sync_copy(nbuf, norm_hbm.at[pl.ds(s * eps, eps)])


_norm_kernel = pl.kernel(
    _norm_body,
    out_type=jax.ShapeDtypeStruct((E,), jnp.float32),
    mesh=_MESH,
    compiler_params=_SC_PARAMS,
    scratch_types=[
        pltpu.VMEM_SHARED((NS * 640,), jnp.float32),
        pltpu.VMEM_SHARED((NS * 640,), jnp.float32),
        pltpu.VMEM((640,), jnp.float32),
        pltpu.VMEM((80,), jnp.float32),
        pltpu.VMEM((E // NS,), jnp.int32),
        pltpu.VMEM((E // NS,), jnp.int32),
        pltpu.VMEM((80,), jnp.int32),
        pltpu.VMEM((80,), jnp.int32),
        pltpu.VMEM((NS * 640,), jnp.float32),
        pltpu.VMEM((E // NS,), jnp.float32),
        pltpu.SemaphoreType.DMA,
        pltpu.SemaphoreType.DMA,
    ],
)


def _pack_chunks(dst, srcidx, val, ch):
    q = dst.shape[0] // ch
    packed = jnp.concatenate([
        dst.reshape(q, ch),
        srcidx.reshape(q, ch),
        lax.bitcast_convert_type(val, jnp.int32).reshape(q, ch),
    ], axis=1)
    return jnp.pad(packed, ((0, 3), (0, 0))).reshape(-1)


_D = 4


def _make_scatter(n_acc, n_entries, n_out_rows, ch, src_stride):
    eps = n_entries // NS
    nch = eps // ch
    arows = n_acc // NS
    assert eps % ch == 0 and (nch - 5) % _D == 0 and nch >= 5
    assert ch % L == 0 and 80 <= ch <= 128 and arows % 80 == 0
    nb = arows // 80

    def body(src_hbm, packed_hbm, out_hbm, acc_sh, *bufs):
        c = lax.axis_index("c")
        s = lax.axis_index("s")
        coff = c * src_stride
        p = bufs[0:_D]
        rv = bufs[_D:2 * _D]
        cv = bufs[2 * _D:3 * _D]
        vv = bufs[3 * _D:4 * _D]
        rows = bufs[4 * _D:5 * _D]
        sp = bufs[5 * _D:6 * _D]
        sg = bufs[6 * _D:7 * _D]
        ss = bufs[7 * _D:8 * _D]

        def zero_row(i, _):
            for j in range(C // L):
                rows[0][i, pl.ds(j * L, L)] = jnp.zeros((L,), jnp.float32)
            return _
        lax.fori_loop(0, 80, zero_row, None)
        zsrc = rows[0] if ch == 80 else rows[0].at[pl.ds(0, 80)]
        for k in range(nb):
            pltpu.async_copy(zsrc, acc_sh.at[pl.ds(s * arows + k * 80, 80)], sg[0])
        for k in range(nb):
            pltpu.make_async_copy(zsrc, acc_sh.at[pl.ds(s * arows + k * 80, 80)], sg[0]).wait()
        plsc.subcore_barrier()

        def pslice(t):
            return packed_hbm.at[pl.ds((s * nch + t) * 3 * ch, 3 * ch)]

        def pstart(t, b):
            pltpu.async_copy(pslice(t), p[b], sp[b])

        def pwait(t, b):
            pltpu.make_async_copy(pslice(t), p[b], sp[b]).wait()

        def gstart(t, b):
            pwait(t, b)
            for g in range(ch // L):
                sl = pl.ds(g * L, L)
                rv[b][sl] = p[b][pl.ds(g * L, L)]
                cv[b][sl] = p[b][pl.ds(ch + g * L, L)] + coff
                vv[b][sl] = plsc.bitcast(p[b][pl.ds(2 * ch + g * L, L)], jnp.float32)
            pltpu.async_copy(src_hbm.at[cv[b]], rows[b], sg[b])
            pstart(t + 2, (b + 2) % _D)

        def gwait(b):
            pltpu.make_async_copy(src_hbm.at[cv[b]], rows[b], sg[b]).wait()

        def scale(b):
            vb, rb = vv[b], rows[b]

            @plsc.parallel_loop(0, ch, 1, unroll=4)
            def srow(i):
                nv = plsc.load_gather(vb, [jnp.full((L,), i, jnp.int32)])
                for j in range(C // L):
                    sl = (i, pl.ds(j * L, L))
                    rb[sl] = rb[sl] * nv

        def sstart(b):
            pltpu.async_copy(rows[b], acc_sh.at[rv[b]], ss[b], add=True)

        def swait(b):
            pltpu.make_async_copy(rows[b], acc_sh.at[rv[b]], ss[b]).wait()

        pstart(0, 0)
        pstart(1, 1)
        gstart(0, 0)
        gstart(1, 1)
        gwait(0); scale(0); sstart(0); gstart(2, 2)
        gwait(1); scale(1); sstart(1); gstart(3, 3)

        def chunk(t, b):
            gwait(b); scale(b); sstart(b); swait((b + 2) % _D)
            gstart(t + 2, (b + 2) % _D)

        def step(g, _):
            t0 = 4 * g + 2
            for o in range(_D):
                chunk(t0 + o, (2 + o) % _D)
            return _
        lax.fori_loop(0, (nch - 5) // _D, step, None)
        t0 = nch - 3
        chunk(t0, 2)
        chunk(t0 + 1, 3)
        gwait(0); scale(0); sstart(0); swait(2)
        swait(3)
        swait(0)
        gwait(1)
        pwait(nch + 1, 2)
        pwait(nch + 2, 3)
        plsc.subcore_barrier()

        r = s * arows
        pltpu.sync_copy(acc_sh.at[pl.ds(r, arows)],
                        out_hbm.at[pl.ds(c * n_out_rows + r, arows)])

    return pl.kernel(
        body,
        out_type=jax.ShapeDtypeStruct((NC * n_out_rows, C), jnp.float32),
        mesh=_MESH,
        compiler_params=_SC_PARAMS,
        scratch_types=(
            [pltpu.VMEM_SHARED((n_acc, C), jnp.float32)]
            + [pltpu.VMEM((3 * ch,), jnp.int32)] * _D
            + [pltpu.VMEM((ch,), jnp.int32)] * _D
            + [pltpu.VMEM((ch,), jnp.int32)] * _D
            + [pltpu.VMEM((ch,), jnp.float32)] * _D
            + [pltpu.VMEM((ch, C), jnp.float32)] * _D
            + [pltpu.SemaphoreType.DMA] * (3 * _D)
        ),
    )


_prop1_kernel = _make_scatter(n_acc=N_PAD, n_entries=E, n_out_rows=N_PAD,
                              ch=80, src_stride=N)
_prop2_kernel = _make_scatter(n_acc=N_PAD, n_entries=E, n_out_rows=N_PAD,
                              ch=80, src_stride=N_PAD)
_pool_kernel = _make_scatter(n_acc=M_PAD, n_entries=NNZ_PAD, n_out_rows=M_PAD,
                             ch=96, src_stride=N)


def _mm_body(x_ref, p1_ref, p2_ref, w_ref, b_ref, o_ref):
    w0 = w_ref[0] - w_ref[2]
    w1 = w_ref[1]
    w2 = 2.0 * w_ref[2]
    z = jnp.dot(x_ref[0], w0, preferred_element_type=jnp.float32)
    z = z + jnp.dot(p1_ref[0], w1, preferred_element_type=jnp.float32)
    z = z + jnp.dot(p2_ref[0], w2, preferred_element_type=jnp.float32)
    z = z + b_ref[...]
    o_ref[0] = jnp.where(z > 0, z, jnp.exp(z) - 1.0)


def _mm_call(x3, p13, p23, W, b):
    B = x3.shape[0]
    blk = 1000
    spec = pl.BlockSpec((1, blk, C), lambda bi, i: (bi, i, 0))
    return pl.pallas_call(
        _mm_body,
        grid=(B, N // blk),
        in_specs=[spec, spec, spec,
                  pl.BlockSpec((3, C, C), lambda bi, i: (0, 0, 0)),
                  pl.BlockSpec((1, C), lambda bi, i: (0, 0))],
        out_specs=spec,
        out_shape=jax.ShapeDtypeStruct((B, N, C), jnp.float32),
    )(x3, p13, p23, W, b)


def kernel(x, edge_index, trans_row, trans_col, trans_value, W, b):
    B = x.shape[0]
    row = edge_index[0]
    col = edge_index[1]

    norm = _norm_kernel(row, col)

    xf = x.reshape(B * N, C)
    epack = _pack_chunks(row, col, norm, 80)
    p1f = _prop1_kernel(xf, epack)
    p2f = _prop2_kernel(p1f, epack)

    h3 = _mm_call(x, p1f.reshape(B, N_PAD, C), p2f.reshape(B, N_PAD, C),
                  W, b.reshape(1, C))
    hf = h3.reshape(B * N, C)

    pad = NNZ_PAD - trans_row.shape[0]
    tr = jnp.concatenate([trans_row, jnp.zeros((pad,), jnp.int32)])
    tc = jnp.concatenate([trans_col, jnp.zeros((pad,), jnp.int32)])
    tv = jnp.concatenate([trans_value, jnp.zeros((pad,), jnp.float32)])

    pooled = _pool_kernel(hf, _pack_chunks(tr, tc, tv, 96))
    return pooled.reshape(B, M_PAD, C)[:, :M, :]
